# Initial kernel scaffold; baseline (speedup 1.0000x reference)
import jax
import jax.numpy as jnp
from jax.experimental import pallas as pl


def kernel(x, adj, adj_ad, W_heads, a_heads, W_out, a_out):
    raise NotImplementedError("write your pallas kernel here")


# speedup vs baseline: 1.6417x; 1.6417x over previous
import functools

import jax
import jax.numpy as jnp
from jax.experimental import pallas as pl
from jax.experimental.pallas import tpu as pltpu

LRELU_ALPHA = 0.2
NEG_BIG = -9e15


def _softmax_rows(att):
    m = jnp.max(att, axis=1, keepdims=True)
    p = jnp.exp(att - m)
    s = jnp.sum(p, axis=1, keepdims=True)
    return p / s


def _leaky(x):
    return jnp.where(x >= 0, x, LRELU_ALPHA * x)


def _elu(x):
    return jnp.where(x > 0, x, jnp.exp(x) - 1.0)


def _layer1_body(x_ref, xT_ref, adj_ref, ad_ref, Wall_ref, WallT_ref,
                 A1_ref, A2T_ref, out_ref, h_ref, f1_ref, f2T_ref,
                 *, nheads, nhid, blk):
    i = pl.program_id(0)

    @pl.when(i == 0)
    def _prologue():
        h = jnp.dot(x_ref[...], Wall_ref[...],
                    preferred_element_type=jnp.float32)
        h_ref[...] = h
        f1_ref[...] = jnp.dot(h, A1_ref[...],
                              preferred_element_type=jnp.float32)
        hT = jnp.dot(WallT_ref[...], xT_ref[...],
                     preferred_element_type=jnp.float32)
        f2T_ref[...] = jnp.dot(A2T_ref[...], hT,
                               preferred_element_type=jnp.float32)

    ad = ad_ref[...]
    mask = adj_ref[...].astype(jnp.int32) > 0
    row0 = i * blk
    for k in range(nheads):
        f1k = f1_ref[pl.ds(row0, blk), k:k + 1]
        f2k = f2T_ref[k:k + 1, :]
        e = _leaky(f1k + f2k) * ad
        att = jnp.where(mask, e, NEG_BIG)
        att = _softmax_rows(att)
        hk = h_ref[:, k * nhid:(k + 1) * nhid]
        hp = jnp.dot(att, hk, preferred_element_type=jnp.float32)
        out_ref[:, k * nhid:(k + 1) * nhid] = _elu(hp)


def _layer2_body(hcat_ref, hcatT_ref, adj_ref, ad_ref, Wout_ref, WoutT_ref,
                 ao1_ref, ao2T_ref, out_ref, h2_ref, f1_ref, f2T_ref,
                 *, blk):
    i = pl.program_id(0)

    @pl.when(i == 0)
    def _prologue():
        h2 = jnp.dot(hcat_ref[...], Wout_ref[...],
                     preferred_element_type=jnp.float32)
        h2_ref[...] = h2
        f1_ref[...] = jnp.dot(h2, ao1_ref[...],
                              preferred_element_type=jnp.float32)
        h2T = jnp.dot(WoutT_ref[...], hcatT_ref[...],
                      preferred_element_type=jnp.float32)
        f2T_ref[...] = jnp.dot(ao2T_ref[...], h2T,
                               preferred_element_type=jnp.float32)

    ad = ad_ref[...]
    mask = adj_ref[...].astype(jnp.int32) > 0
    row0 = i * blk
    f1k = f1_ref[pl.ds(row0, blk), :]
    e = _leaky(f1k + f2T_ref[...]) * ad
    att = jnp.where(mask, e, NEG_BIG)
    att = _softmax_rows(att)
    hp = jnp.dot(att, h2_ref[...], preferred_element_type=jnp.float32)
    g = _elu(hp)
    m = jnp.max(g, axis=1, keepdims=True)
    z = g - m
    lse = jnp.log(jnp.sum(jnp.exp(z), axis=1, keepdims=True))
    out_ref[...] = z - lse


def kernel(x, adj, adj_ad, W_heads, a_heads, W_out, a_out):
    n, nfeat = x.shape
    nheads, _, nhid = W_heads.shape
    hd = nheads * nhid
    nclass = W_out.shape[1]
    blk = min(256, n)
    nb = n // blk

    adj8 = adj.astype(jnp.int8)
    xT = x.T
    W_all = jnp.transpose(W_heads, (1, 0, 2)).reshape(nfeat, hd)
    W_allT = W_all.T
    eye = jnp.eye(nheads, dtype=x.dtype)
    a1 = a_heads[:, :nhid, 0]
    a2 = a_heads[:, nhid:, 0]
    A1 = jnp.reshape(a1[:, :, None] * eye[:, None, :], (hd, nheads))
    A2T = jnp.reshape(a2[:, :, None] * eye[:, None, :], (hd, nheads)).T
    W_outT = W_out.T
    ao1 = a_out[:nclass]
    ao2T = a_out[nclass:].T

    full = lambda shape: pl.BlockSpec(shape, lambda i: (0, 0))
    rows = lambda width: pl.BlockSpec((blk, width), lambda i: (i, 0))

    hcat = pl.pallas_call(
        functools.partial(_layer1_body, nheads=nheads, nhid=nhid, blk=blk),
        grid=(nb,),
        in_specs=[
            full((n, nfeat)), full((nfeat, n)),
            rows(n), rows(n),
            full((nfeat, hd)), full((hd, nfeat)),
            full((hd, nheads)), full((nheads, hd)),
        ],
        out_specs=rows(hd),
        out_shape=jax.ShapeDtypeStruct((n, hd), jnp.float32),
        scratch_shapes=[
            pltpu.VMEM((n, hd), jnp.float32),
            pltpu.VMEM((n, nheads), jnp.float32),
            pltpu.VMEM((nheads, n), jnp.float32),
        ],
    )(x, xT, adj8, adj_ad, W_all, W_allT, A1, A2T)

    hcatT = hcat.T

    out = pl.pallas_call(
        functools.partial(_layer2_body, blk=blk),
        grid=(nb,),
        in_specs=[
            full((n, hd)), full((hd, n)),
            rows(n), rows(n),
            full((hd, nclass)), full((nclass, hd)),
            full((nclass, 1)), full((1, nclass)),
        ],
        out_specs=rows(nclass),
        out_shape=jax.ShapeDtypeStruct((n, nclass), jnp.float32),
        scratch_shapes=[
            pltpu.VMEM((n, nclass), jnp.float32),
            pltpu.VMEM((n, 1), jnp.float32),
            pltpu.VMEM((1, n), jnp.float32),
        ],
    )(hcat, hcatT, adj8, adj_ad, W_out, W_outT, ao1, ao2T)

    return out
